# baseline (speedup 1.0000x reference)
import functools
import math

import jax
import jax.numpy as jnp
from jax import lax
from jax.experimental import pallas as pl
from jax.experimental.pallas import tpu as pltpu
from jax.experimental.pallas import tpu_sc as plsc

_D = 64
_SCALE = 8.0
_LANES = 16
_NC = 2
_NS = 16
_NW = _NC * _NS
_C = 128


@jax.jit
def _gather_scale(idx, lut):
    B = idx.shape[0]
    bpw = B // _NW
    nchunk = bpw // _C
    mesh = plsc.VectorSubcoreMesh(core_axis_name="c", subcore_axis_name="s")

    @functools.partial(
        pl.kernel,
        out_type=jax.ShapeDtypeStruct((B, _D), jnp.float32),
        mesh=mesh,
        scratch_types=[
            pltpu.VMEM((bpw,), jnp.int32),
            pltpu.VMEM((_C, _D), jnp.float32),
            pltpu.SemaphoreType.DMA,
        ],
        compiler_params=pltpu.CompilerParams(use_tc_tiling_on_sc=False),
    )
    def body(idx_hbm, lut_hbm, out_hbm, idx_v, buf, gsem):
        wid = lax.axis_index("s") * _NC + lax.axis_index("c")
        base = wid * bpw
        pltpu.sync_copy(idx_hbm.at[pl.ds(base, bpw)], idx_v)

        def chunk(g, carry):
            pltpu.async_copy(
                lut_hbm.at[idx_v.at[pl.ds(g * _C, _C)]], buf, gsem
            ).wait()

            def row(i, c2):
                for j in range(_D // _LANES):
                    sl = buf[i, pl.ds(j * _LANES, _LANES)]
                    buf[i, pl.ds(j * _LANES, _LANES)] = sl * _SCALE
                return c2

            lax.fori_loop(0, _C, row, 0)
            pltpu.sync_copy(buf, out_hbm.at[pl.ds(base + g * _C, _C)])
            return carry

        lax.fori_loop(0, nchunk, chunk, 0)

    return body(idx, lut)


def kernel(x, lut):
    r, s = x.shape
    idx = x.reshape(r * s).astype(jnp.int32)
    out = _gather_scale(idx, lut)
    return out.reshape(r, s, _D)


# speedup vs baseline: 1.0802x; 1.0802x over previous
import functools
import math

import jax
import jax.numpy as jnp
from jax import lax
from jax.experimental import pallas as pl
from jax.experimental.pallas import tpu as pltpu
from jax.experimental.pallas import tpu_sc as plsc

_D = 64
_SCALE = 8.0
_LANES = 16
_NC = 2
_NS = 16
_NW = _NC * _NS
_C = 128
_NBUF = 5
_LOOK = 3


@jax.jit
def _gather_scale(idx, lut):
    B = idx.shape[0]
    bpw = B // _NW
    nchunk = bpw // _C
    ngroup = nchunk // _NBUF
    mesh = plsc.VectorSubcoreMesh(core_axis_name="c", subcore_axis_name="s")

    @functools.partial(
        pl.kernel,
        out_type=jax.ShapeDtypeStruct((B, _D), jnp.float32),
        mesh=mesh,
        scratch_types=[
            pltpu.VMEM((bpw,), jnp.int32),
            pltpu.VMEM((_NBUF, _C, _D), jnp.float32),
            pltpu.SemaphoreType.DMA((_NBUF,)),
            pltpu.SemaphoreType.DMA((_NBUF,)),
        ],
        compiler_params=pltpu.CompilerParams(use_tc_tiling_on_sc=False),
    )
    def body(idx_hbm, lut_hbm, out_hbm, idx_v, bufs, gsem, ssem):
        wid = lax.axis_index("s") * _NC + lax.axis_index("c")
        base = wid * bpw
        pltpu.sync_copy(idx_hbm.at[pl.ds(base, bpw)], idx_v)

        def fire_gather(g, b):
            pltpu.async_copy(
                lut_hbm.at[idx_v.at[pl.ds(g * _C, _C)]],
                bufs.at[b],
                gsem.at[b],
            )

        def wait_gather(b):
            pltpu.make_async_copy(
                lut_hbm.at[idx_v.at[pl.ds(0, _C)]], bufs.at[b], gsem.at[b]
            ).wait()

        def fire_scatter(g, b):
            pltpu.async_copy(
                bufs.at[b], out_hbm.at[pl.ds(base + g * _C, _C)], ssem.at[b]
            )

        def wait_scatter(b):
            pltpu.make_async_copy(
                bufs.at[b], out_hbm.at[pl.ds(base, _C)], ssem.at[b]
            ).wait()

        for b in range(_LOOK):
            fire_gather(b, b)

        def group(go, carry):
            for b in range(_NBUF):
                g = go * _NBUF + b
                p = g + _LOOK
                pb = (b + _LOOK) % _NBUF

                @pl.when(jnp.logical_and(p >= _NBUF, p < nchunk))
                def _():
                    wait_scatter(pb)

                @pl.when(p < nchunk)
                def _():
                    fire_gather(p, pb)

                wait_gather(b)

                def row(i, c2):
                    for j in range(_D // _LANES):
                        sl = bufs[b, i, pl.ds(j * _LANES, _LANES)]
                        bufs[b, i, pl.ds(j * _LANES, _LANES)] = sl * _SCALE
                    return c2

                lax.fori_loop(0, _C, row, 0)
                fire_scatter(g, b)
            return carry

        lax.fori_loop(0, ngroup, group, 0)

        for b in range(_NBUF):
            wait_scatter(b)

    return body(idx, lut)


def kernel(x, lut):
    r, s = x.shape
    idx = x.reshape(r * s).astype(jnp.int32)
    out = _gather_scale(idx, lut)
    return out.reshape(r, s, _D)


# speedup vs baseline: 1.1008x; 1.0190x over previous
import functools
import math

import jax
import jax.numpy as jnp
from jax import lax
from jax.experimental import pallas as pl
from jax.experimental.pallas import tpu as pltpu
from jax.experimental.pallas import tpu_sc as plsc

_D = 64
_SCALE = 8.0
_LANES = 16
_NC = 2
_NS = 16
_NW = _NC * _NS
_C = 128
_NBUF = 5
_LOOK = 3


@jax.jit
def _gather_scale(idx2d, lut):
    S, B = idx2d.shape
    nchunk = S
    ngroup = nchunk // _NBUF
    mesh = plsc.VectorSubcoreMesh(core_axis_name="c", subcore_axis_name="s")

    @functools.partial(
        pl.kernel,
        out_type=jax.ShapeDtypeStruct((S * B, _D), jnp.float32),
        mesh=mesh,
        scratch_types=[
            pltpu.VMEM((S, _C), jnp.int32),
            pltpu.VMEM((_NBUF, _C, _D), jnp.float32),
            pltpu.SemaphoreType.DMA((_NBUF,)),
            pltpu.SemaphoreType.DMA((_NBUF,)),
            pltpu.SemaphoreType.DMA,
        ],
        compiler_params=pltpu.CompilerParams(use_tc_tiling_on_sc=False),
    )
    def body(idx_hbm, lut_hbm, out_hbm, idx_v, bufs, gsem, ssem, isem):
        wid = lax.axis_index("s") * _NC + lax.axis_index("c")
        col0 = wid * _C

        pltpu.async_copy(
            idx_hbm.at[:, pl.ds(col0, _C)], idx_v, isem
        ).wait()

        def fire_gather(g, b):
            pltpu.async_copy(
                lut_hbm.at[idx_v.at[g]], bufs.at[b], gsem.at[b]
            )

        def wait_gather(b):
            pltpu.make_async_copy(
                lut_hbm.at[idx_v.at[0]], bufs.at[b], gsem.at[b]
            ).wait()

        def fire_scatter(g, b):
            pltpu.async_copy(
                bufs.at[b],
                out_hbm.at[pl.ds((g * (B // _C) + wid) * _C, _C)],
                ssem.at[b],
            )

        def wait_scatter(b):
            pltpu.make_async_copy(
                bufs.at[b], out_hbm.at[pl.ds(0, _C)], ssem.at[b]
            ).wait()

        for b in range(_LOOK):
            fire_gather(b, b)

        def group(go, carry):
            for b in range(_NBUF):
                g = go * _NBUF + b
                p = g + _LOOK
                pb = (b + _LOOK) % _NBUF

                @pl.when(jnp.logical_and(p >= _NBUF, p < nchunk))
                def _():
                    wait_scatter(pb)

                @pl.when(p < nchunk)
                def _():
                    fire_gather(p, pb)

                wait_gather(b)

                def row(i, c2):
                    for j in range(_D // _LANES):
                        sl = bufs[b, i, pl.ds(j * _LANES, _LANES)]
                        bufs[b, i, pl.ds(j * _LANES, _LANES)] = sl * _SCALE
                    return c2

                lax.fori_loop(0, _C, row, 0)
                fire_scatter(g, b)
            return carry

        lax.fori_loop(0, ngroup, group, 0)

        for b in range(_NBUF):
            wait_scatter(b)

    return body(idx2d, lut)


def kernel(x, lut):
    r, s = x.shape
    idx2d = x.T.astype(jnp.int32)
    out2 = _gather_scale(idx2d, lut)
    return jnp.transpose(out2.reshape(s, r, _D), (1, 0, 2))


# speedup vs baseline: 1.6257x; 1.4769x over previous
import functools
import math

import jax
import jax.numpy as jnp
from jax import lax
from jax.experimental import pallas as pl
from jax.experimental.pallas import tpu as pltpu
from jax.experimental.pallas import tpu_sc as plsc

_D = 64
_SCALE = 8.0
_LANES = 16
_NC = 2
_NS = 16
_NW = _NC * _NS
_C = 128
_NBUF = 5
_LOOK = 3


@jax.jit
def _gather_scale(idx1d, lut):
    B = 4096
    S = idx1d.shape[0] // B
    nchunk = S
    ngroup = nchunk // _NBUF
    mesh = plsc.VectorSubcoreMesh(core_axis_name="c", subcore_axis_name="s")

    @functools.partial(
        pl.kernel,
        out_type=jax.ShapeDtypeStruct((S * B, _D), jnp.float32),
        mesh=mesh,
        scratch_types=[
            pltpu.VMEM((_NBUF, _C), jnp.int32),
            pltpu.VMEM((_NBUF, _C, _D), jnp.float32),
            pltpu.SemaphoreType.DMA((_NBUF,)),
            pltpu.SemaphoreType.DMA((_NBUF,)),
            pltpu.SemaphoreType.DMA((_NBUF,)),
        ],
        compiler_params=pltpu.CompilerParams(use_tc_tiling_on_sc=True),
    )
    def body(idx_hbm, lut_hbm, out_hbm, idx_v, bufs, isem, gsem, ssem):
        wid = lax.axis_index("s") * _NC + lax.axis_index("c")
        col0 = wid * _C

        def fire_idx(g, b):
            pltpu.async_copy(
                idx_hbm.at[pl.ds(g * B + col0, _C)], idx_v.at[b], isem.at[b]
            )

        def wait_idx(b):
            pltpu.make_async_copy(
                idx_hbm.at[pl.ds(0, _C)], idx_v.at[b], isem.at[b]
            ).wait()

        def fire_gather(b):
            def grp(t, carry):
                vec = idx_v[b, pl.ds(t * _LANES, _LANES)]
                for i2 in range(_LANES):
                    r = vec[i2]
                    pltpu.async_copy(
                        lut_hbm.at[pl.ds(r, 1)],
                        bufs.at[b, pl.ds(t * _LANES + i2, 1)],
                        gsem.at[b],
                    )
                return carry

            lax.fori_loop(0, _C // _LANES, grp, 0)

        def wait_gather(b):
            pltpu.make_async_copy(
                lut_hbm.at[pl.ds(0, _C)], bufs.at[b], gsem.at[b]
            ).wait()

        def fire_scatter(g, b):
            pltpu.async_copy(
                bufs.at[b],
                out_hbm.at[pl.ds((g * (B // _C) + wid) * _C, _C)],
                ssem.at[b],
            )

        def wait_scatter(b):
            pltpu.make_async_copy(
                bufs.at[b], out_hbm.at[pl.ds(0, _C)], ssem.at[b]
            ).wait()

        for b in range(_LOOK):
            fire_idx(b, b)

        def group(go, carry):
            for b in range(_NBUF):
                g = go * _NBUF + b
                p = g + _LOOK
                pb = (b + _LOOK) % _NBUF

                @pl.when(jnp.logical_and(p >= _NBUF, p < nchunk))
                def _():
                    wait_scatter(pb)

                @pl.when(p < nchunk)
                def _():
                    fire_idx(p, pb)

                wait_idx(b)
                fire_gather(b)
                wait_gather(b)

                def row(i, c2):
                    for j in range(_D // _LANES):
                        sl = bufs[b, i, pl.ds(j * _LANES, _LANES)]
                        bufs[b, i, pl.ds(j * _LANES, _LANES)] = sl * _SCALE
                    return c2

                lax.fori_loop(0, _C, row, 0)
                fire_scatter(g, b)
            return carry

        lax.fori_loop(0, ngroup, group, 0)

        for b in range(_NBUF):
            wait_scatter(b)

    return body(idx1d, lut)


def kernel(x, lut):
    r, s = x.shape
    idx1d = x.T.reshape(r * s).astype(jnp.int32)
    out2 = _gather_scale(idx1d, lut)
    return jnp.transpose(out2.reshape(s, r, _D), (1, 0, 2))


# speedup vs baseline: 2.4350x; 1.4977x over previous
import functools
import math

import jax
import jax.numpy as jnp
from jax import lax
from jax.experimental import pallas as pl
from jax.experimental.pallas import tpu as pltpu
from jax.experimental.pallas import tpu_sc as plsc

_D = 64
_SCALE = 8.0
_LANES = 16
_NC = 2
_NS = 16
_NW = _NC * _NS
_C = 128
_NBUF = 5
_LOOK = 3


@jax.jit
def _gather_scale(idx1d, lut3):
    B = 4096
    S = idx1d.shape[0] // B
    nchunk = S
    ngroup = nchunk // _NBUF
    mesh = plsc.VectorSubcoreMesh(core_axis_name="c", subcore_axis_name="s")

    @functools.partial(
        pl.kernel,
        out_type=jax.ShapeDtypeStruct((S * B, _D), jnp.float32),
        mesh=mesh,
        scratch_types=[
            pltpu.VMEM((_NBUF, _C), jnp.int32),
            pltpu.VMEM((_NBUF, _C, _D), jnp.float32),
            pltpu.SemaphoreType.DMA((_NBUF,)),
            pltpu.SemaphoreType.DMA((_NBUF,)),
            pltpu.SemaphoreType.DMA((_NBUF,)),
        ],
        compiler_params=pltpu.CompilerParams(use_tc_tiling_on_sc=True),
    )
    def body(idx_hbm, lut_hbm, out_hbm, idx_v, bufs, isem, gsem, ssem):
        wid = lax.axis_index("s") * _NC + lax.axis_index("c")
        col0 = wid * _C

        def fire_idx(g, b):
            pltpu.async_copy(
                idx_hbm.at[pl.ds(g * B + col0, _C)], idx_v.at[b], isem.at[b]
            )

        def wait_idx(b):
            pltpu.make_async_copy(
                idx_hbm.at[pl.ds(0, _C)], idx_v.at[b], isem.at[b]
            ).wait()

        def fire_gather(b):
            def grp(t, carry):
                vec = idx_v[b, pl.ds(t * _LANES, _LANES)]
                for i2 in range(_LANES):
                    r = vec[i2]
                    pltpu.async_copy(
                        lut_hbm.at[r >> 3, r & 7],
                        bufs.at[b, t * _LANES + i2],
                        gsem.at[b],
                    )
                return carry

            lax.fori_loop(0, _C // _LANES, grp, 0)

        def wait_gather(b):
            pltpu.make_async_copy(
                out_hbm.at[pl.ds(0, _C)], bufs.at[b], gsem.at[b]
            ).wait()

        def fire_scatter(g, b):
            pltpu.async_copy(
                bufs.at[b],
                out_hbm.at[pl.ds((g * (B // _C) + wid) * _C, _C)],
                ssem.at[b],
            )

        def wait_scatter(b):
            pltpu.make_async_copy(
                bufs.at[b], out_hbm.at[pl.ds(0, _C)], ssem.at[b]
            ).wait()

        for b in range(_LOOK):
            fire_idx(b, b)
        wait_idx(0)
        fire_gather(0)

        def group(go, carry):
            for b in range(_NBUF):
                g = go * _NBUF + b
                p = g + _LOOK
                pbi = (b + _LOOK) % _NBUF
                c = g + 1
                bc = (b + 1) % _NBUF

                @pl.when(p < nchunk)
                def _():
                    fire_idx(p, pbi)

                @pl.when(jnp.logical_and(c >= _NBUF, c < nchunk))
                def _():
                    wait_scatter(bc)

                @pl.when(c < nchunk)
                def _():
                    wait_idx(bc)
                    fire_gather(bc)

                wait_gather(b)

                def row(i, c2):
                    for j in range(_D // _LANES):
                        sl = bufs[b, i, pl.ds(j * _LANES, _LANES)]
                        bufs[b, i, pl.ds(j * _LANES, _LANES)] = sl * _SCALE
                    return c2

                lax.fori_loop(0, _C, row, 0)
                fire_scatter(g, b)
            return carry

        lax.fori_loop(0, ngroup, group, 0)

        for b in range(_NBUF):
            wait_scatter(b)

    return body(idx1d, lut3)


def kernel(x, lut):
    r, s = x.shape
    idx1d = x.T.reshape(r * s).astype(jnp.int32)
    lut3 = lut.reshape(lut.shape[0] // 8, 8, _D)
    out2 = _gather_scale(idx1d, lut3)
    return jnp.transpose(out2.reshape(s, r, _D), (1, 0, 2))
